# Initial kernel scaffold; baseline (speedup 1.0000x reference)
import jax
import jax.numpy as jnp
from jax.experimental import pallas as pl


def kernel(x, boundaries, table):
    raise NotImplementedError("write your pallas kernel here")


# speedup vs baseline: 68.3677x; 68.3677x over previous
import functools

import jax
import jax.numpy as jnp
from jax import lax
from jax.experimental import pallas as pl
from jax.experimental.pallas import tpu as pltpu
from jax.experimental.pallas import tpu_sc as plsc

N_BINS = 1000
HIDDEN = 64
MIN_VAL = -5.0
SCALE = 100.0

_info = plsc.get_sparse_core_info()
_NC, _NS = _info.num_cores, _info.num_subcores
_NW = _NC * _NS

CHUNK = 1024
_IDX_ROWS, _IDX_COLS = 8, 128


def _make_sc_call(total, hp_len):
    per_w = total // _NW
    n_chunks = per_w // CHUNK
    x_rows_per_chunk = CHUNK // _IDX_COLS

    mesh = plsc.VectorSubcoreMesh(core_axis_name="c", subcore_axis_name="s")

    @functools.partial(
        pl.kernel,
        mesh=mesh,
        compiler_params=pltpu.CompilerParams(
            needs_layout_passes=False, use_tc_tiling_on_sc=False
        ),
        out_type=jax.ShapeDtypeStruct((total, HIDDEN), jnp.float32),
        scratch_types=[
            pltpu.VMEM((_IDX_ROWS, _IDX_COLS), jnp.float32),
            pltpu.VMEM((_IDX_ROWS, _IDX_COLS), jnp.int32),
            pltpu.VMEM((CHUNK, HIDDEN), jnp.float32),
            pltpu.VMEM((hp_len,), jnp.float32),
            pltpu.SemaphoreType.DMA,
        ],
    )
    def sc_kernel(x_hbm, hp_hbm, table_hbm, out_hbm, x_v, idx_v, rows_v, hp_v, sem):
        wid = lax.axis_index("s") * _NC + lax.axis_index("c")
        wbase = wid * per_w

        pltpu.sync_copy(hp_hbm, hp_v)

        def chunk_body(t, carry):
            ebase = pl.multiple_of(wbase + t * CHUNK, CHUNK)
            rbase = pl.multiple_of(ebase // _IDX_COLS, 8)
            pltpu.sync_copy(x_hbm.at[pl.ds(rbase, x_rows_per_chunk)], x_v)

            for row in range(_IDX_ROWS):
                for c in range(_IDX_COLS // 16):
                    xv = x_v[row, pl.ds(c * 16, 16)]
                    t0 = (xv - MIN_VAL) * SCALE
                    g = jnp.clip(t0.astype(jnp.int32), 0, N_BINS - 1)
                    hi = plsc.load_gather(hp_v, [g + 1])
                    lo = plsc.load_gather(hp_v, [g])
                    g = g + jnp.where(xv > hi, 1, 0) - jnp.where(xv <= lo, 1, 0)
                    idx_v[row, pl.ds(c * 16, 16)] = g

            copies = []
            for j in range(_IDX_ROWS):
                copies.append(
                    pltpu.async_copy(
                        table_hbm.at[idx_v.at[j]],
                        rows_v.at[pl.ds(j * _IDX_COLS, _IDX_COLS)],
                        sem,
                    )
                )
            for cp in copies:
                cp.wait()

            pltpu.sync_copy(rows_v, out_hbm.at[pl.ds(ebase, CHUNK)])
            return carry

        lax.fori_loop(0, n_chunks, chunk_body, 0)

    return sc_kernel


def kernel(x, boundaries, table):
    total = x.shape[0] * x.shape[1]
    xf = x.reshape(total // _IDX_COLS, _IDX_COLS)
    hp_len = N_BINS + 8
    hp = jnp.concatenate(
        [
            jnp.full((1,), -jnp.inf, jnp.float32),
            boundaries.astype(jnp.float32),
            jnp.full((hp_len - 1 - boundaries.shape[0],), jnp.inf, jnp.float32),
        ]
    )
    out = _make_sc_call(total, hp_len)(xf, hp, table)
    return out.reshape(x.shape[0], x.shape[1], HIDDEN)


# speedup vs baseline: 68.5451x; 1.0026x over previous
import functools

import jax
import jax.numpy as jnp
from jax import lax
from jax.experimental import pallas as pl
from jax.experimental.pallas import tpu as pltpu
from jax.experimental.pallas import tpu_sc as plsc

N_BINS = 1000
HIDDEN = 64
MIN_VAL = -5.0
SCALE = 100.0

_info = plsc.get_sparse_core_info()
_NC, _NS = _info.num_cores, _info.num_subcores
_NW = _NC * _NS

CHUNK = 640
_ROWS = CHUNK // 128


def _make_sc_call(total, hp_len):
    per_w = total // _NW
    n_chunks = per_w // CHUNK
    n_groups = n_chunks // 2

    mesh = plsc.VectorSubcoreMesh(core_axis_name="c", subcore_axis_name="s")

    @functools.partial(
        pl.kernel,
        mesh=mesh,
        compiler_params=pltpu.CompilerParams(
            needs_layout_passes=False, use_tc_tiling_on_sc=False
        ),
        out_type=jax.ShapeDtypeStruct((total, HIDDEN), jnp.float32),
        scratch_types=[
            pltpu.VMEM((2, CHUNK), jnp.float32),
            pltpu.VMEM((2, _ROWS, 128), jnp.int32),
            pltpu.VMEM((2, CHUNK, HIDDEN), jnp.float32),
            pltpu.VMEM((hp_len,), jnp.float32),
            pltpu.SemaphoreType.DMA,
            pltpu.SemaphoreType.DMA,
            pltpu.SemaphoreType.DMA,
        ],
    )
    def sc_kernel(x_hbm, hp_hbm, table_hbm, out_hbm, x_v, idx_v, rows_v, hp_v,
                  xsem, gsem, ssem):
        wid = lax.axis_index("s") * _NC + lax.axis_index("c")
        wbase = wid * per_w

        pltpu.sync_copy(hp_hbm, hp_v)
        pltpu.async_copy(
            x_hbm.at[pl.ds(pl.multiple_of(wbase, CHUNK), CHUNK)], x_v.at[0], xsem
        )

        def cbase(t):
            return pl.multiple_of(wbase + t * CHUNK, CHUNK)

        def wait_store(b):
            pltpu.make_async_copy(
                rows_v.at[b], out_hbm.at[pl.ds(0, CHUNK)], ssem
            ).wait()

        def wait_gathers_fire_store(bb, ebase_prev):
            pltpu.make_async_copy(
                out_hbm.at[pl.ds(0, CHUNK)], rows_v.at[bb], gsem
            ).wait()
            pltpu.async_copy(rows_v.at[bb], out_hbm.at[pl.ds(ebase_prev, CHUNK)], ssem)

        def group_body(g, carry):
            for b in range(2):
                t = g * 2 + b
                ebase = cbase(t)

                if b == 0:
                    pl.when(g >= 1)(lambda: wait_store(0))
                else:
                    pl.when(g >= 1)(lambda: wait_store(1))

                pltpu.make_async_copy(
                    x_hbm.at[pl.ds(0, CHUNK)], x_v.at[b], xsem
                ).wait()

                def prefetch():
                    pltpu.async_copy(
                        x_hbm.at[pl.ds(cbase(t + 1), CHUNK)], x_v.at[1 - b], xsem
                    )
                if b == 0:
                    prefetch()
                else:
                    pl.when(g < n_groups - 1)(prefetch)

                for r in range(_ROWS):
                    for c in range(8):
                        off = r * 128 + c * 16
                        xv = x_v[b, pl.ds(off, 16)]
                        t0 = (xv - MIN_VAL) * SCALE
                        gi = jnp.clip(t0.astype(jnp.int32), 0, N_BINS - 1)
                        hi = plsc.load_gather(hp_v, [gi + 1])
                        lo = plsc.load_gather(hp_v, [gi])
                        gi = gi + jnp.where(xv > hi, 1, 0) - jnp.where(xv <= lo, 1, 0)
                        idx_v[b, r, pl.ds(c * 16, 16)] = gi

                for r in range(_ROWS):
                    pltpu.async_copy(
                        table_hbm.at[idx_v.at[b, r]],
                        rows_v.at[b, pl.ds(r * 128, 128)],
                        gsem,
                    )

                if b == 1:
                    wait_gathers_fire_store(0, cbase(t - 1))
                else:
                    pl.when(g >= 1)(
                        functools.partial(wait_gathers_fire_store, 1, cbase(t - 1))
                    )
            return carry

        lax.fori_loop(0, n_groups, group_body, 0)

        last = n_chunks - 1
        pltpu.make_async_copy(
            out_hbm.at[pl.ds(0, CHUNK)], rows_v.at[1], gsem
        ).wait()
        pltpu.async_copy(rows_v.at[1], out_hbm.at[pl.ds(cbase(last), CHUNK)], ssem)
        wait_store(0)
        wait_store(1)

    return sc_kernel


def kernel(x, boundaries, table):
    total = x.shape[0] * x.shape[1]
    xf = x.reshape(total)
    hp_len = N_BINS + 8
    hp = jnp.concatenate(
        [
            jnp.full((1,), -jnp.inf, jnp.float32),
            boundaries.astype(jnp.float32),
            jnp.full((hp_len - 1 - boundaries.shape[0],), jnp.inf, jnp.float32),
        ]
    )
    out = _make_sc_call(total, hp_len)(xf, hp, table)
    return out.reshape(x.shape[0], x.shape[1], HIDDEN)


# speedup vs baseline: 68.7559x; 1.0031x over previous
import functools

import jax
import jax.numpy as jnp
from jax import lax
from jax.experimental import pallas as pl
from jax.experimental.pallas import tpu as pltpu
from jax.experimental.pallas import tpu_sc as plsc

N_BINS = 1000
HIDDEN = 64
MIN_VAL = -5.0
SCALE = 100.0

_info = plsc.get_sparse_core_info()
_NC, _NS = _info.num_cores, _info.num_subcores
_NW = _NC * _NS

ROWS_PER_CHUNK = 2


def _make_sc_call(n_rows, row_len, hp_len):
    chunk = ROWS_PER_CHUNK * row_len
    rows_per_w = n_rows // _NW
    n_chunks = rows_per_w // ROWS_PER_CHUNK
    n_groups = n_chunks // 2
    splits = [(0, 128), (128, row_len - 128)]

    mesh = plsc.VectorSubcoreMesh(core_axis_name="c", subcore_axis_name="s")

    @functools.partial(
        pl.kernel,
        mesh=mesh,
        compiler_params=pltpu.CompilerParams(
            needs_layout_passes=False, use_tc_tiling_on_sc=False
        ),
        out_type=jax.ShapeDtypeStruct((n_rows, row_len, HIDDEN), jnp.float32),
        scratch_types=[
            pltpu.VMEM((2, chunk), jnp.float32),
            pltpu.VMEM((2, chunk), jnp.int32),
            pltpu.VMEM((2, ROWS_PER_CHUNK, row_len, HIDDEN), jnp.float32),
            pltpu.VMEM((hp_len,), jnp.float32),
            pltpu.SemaphoreType.DMA,
            pltpu.SemaphoreType.DMA,
            pltpu.SemaphoreType.DMA,
        ],
    )
    def sc_kernel(x_hbm, hp_hbm, table_hbm, out_hbm, x_v, idx_v, rows_v, hp_v,
                  xsem, gsem, ssem):
        wid = lax.axis_index("s") * _NC + lax.axis_index("c")
        wrow = wid * rows_per_w
        webase = wid * rows_per_w * row_len

        pltpu.sync_copy(hp_hbm, hp_v)
        pltpu.async_copy(
            x_hbm.at[pl.ds(pl.multiple_of(webase, chunk), chunk)], x_v.at[0], xsem
        )

        def ebase(t):
            return pl.multiple_of(webase + t * chunk, chunk)

        def rowbase(t):
            return pl.multiple_of(wrow + t * ROWS_PER_CHUNK, ROWS_PER_CHUNK)

        def wait_store(b):
            pltpu.make_async_copy(
                rows_v.at[b], out_hbm.at[pl.ds(0, ROWS_PER_CHUNK)], ssem
            ).wait()

        def wait_gathers_fire_store(bb, t_prev):
            pltpu.make_async_copy(
                out_hbm.at[pl.ds(0, ROWS_PER_CHUNK)], rows_v.at[bb], gsem
            ).wait()
            pltpu.async_copy(
                rows_v.at[bb], out_hbm.at[pl.ds(rowbase(t_prev), ROWS_PER_CHUNK)], ssem
            )

        def group_body(g, carry):
            for b in range(2):
                t = g * 2 + b

                pl.when(g >= 1)(functools.partial(wait_store, b))

                pltpu.make_async_copy(
                    x_hbm.at[pl.ds(0, chunk)], x_v.at[b], xsem
                ).wait()

                def prefetch():
                    pltpu.async_copy(
                        x_hbm.at[pl.ds(ebase(t + 1), chunk)], x_v.at[1 - b], xsem
                    )
                if b == 0:
                    prefetch()
                else:
                    pl.when(g < n_groups - 1)(prefetch)

                for blk in range(chunk // 16):
                    off = blk * 16
                    xv = x_v[b, pl.ds(off, 16)]
                    t0 = (xv - MIN_VAL) * SCALE
                    gi = jnp.clip(t0.astype(jnp.int32), 0, N_BINS - 1)
                    hi = plsc.load_gather(hp_v, [gi + 1])
                    lo = plsc.load_gather(hp_v, [gi])
                    gi = gi + jnp.where(xv > hi, 1, 0) - jnp.where(xv <= lo, 1, 0)
                    idx_v[b, pl.ds(off, 16)] = gi

                for i in range(ROWS_PER_CHUNK):
                    for (soff, slen) in splits:
                        pltpu.async_copy(
                            table_hbm.at[idx_v.at[b, pl.ds(i * row_len + soff, slen)]],
                            rows_v.at[b, i, pl.ds(soff, slen)],
                            gsem,
                        )

                if b == 1:
                    wait_gathers_fire_store(0, t - 1)
                else:
                    pl.when(g >= 1)(
                        functools.partial(wait_gathers_fire_store, 1, t - 1)
                    )
            return carry

        lax.fori_loop(0, n_groups, group_body, 0)

        pltpu.make_async_copy(
            out_hbm.at[pl.ds(0, ROWS_PER_CHUNK)], rows_v.at[1], gsem
        ).wait()
        pltpu.async_copy(
            rows_v.at[1],
            out_hbm.at[pl.ds(rowbase(n_chunks - 1), ROWS_PER_CHUNK)],
            ssem,
        )
        wait_store(0)
        wait_store(1)

    return sc_kernel


def kernel(x, boundaries, table):
    n_rows, row_len = x.shape
    xf = x.reshape(n_rows * row_len)
    hp_len = N_BINS + 8
    hp = jnp.concatenate(
        [
            jnp.full((1,), -jnp.inf, jnp.float32),
            boundaries.astype(jnp.float32),
            jnp.full((hp_len - 1 - boundaries.shape[0],), jnp.inf, jnp.float32),
        ]
    )
    return _make_sc_call(n_rows, row_len, hp_len)(xf, hp, table)


# speedup vs baseline: 101.4327x; 1.4753x over previous
import functools

import jax
import jax.numpy as jnp
from jax import lax
from jax.experimental import pallas as pl
from jax.experimental.pallas import tpu as pltpu
from jax.experimental.pallas import tpu_sc as plsc

N_BINS = 1000
HIDDEN = 64
MIN_VAL = -5.0
SCALE = 100.0
HP_LEN = 1024

_info = plsc.get_sparse_core_info()
_NC, _NS = _info.num_cores, _info.num_subcores
_NW = _NC * _NS

CHUNK = 128


def _make_sc_call(total):
    per_w = total // _NW
    n_chunks = per_w // CHUNK
    n_groups = n_chunks // 2

    mesh = plsc.VectorSubcoreMesh(core_axis_name="c", subcore_axis_name="s")

    @functools.partial(
        pl.kernel,
        mesh=mesh,
        compiler_params=pltpu.CompilerParams(
            needs_layout_passes=False, use_tc_tiling_on_sc=True
        ),
        out_type=jax.ShapeDtypeStruct((total, HIDDEN), jnp.float32),
        scratch_types=[
            pltpu.VMEM((2 * CHUNK,), jnp.float32),
            pltpu.VMEM((2, CHUNK, HIDDEN), jnp.float32),
            pltpu.VMEM((N_BINS * HIDDEN,), jnp.float32),
            pltpu.VMEM((HP_LEN,), jnp.float32),
            pltpu.SemaphoreType.DMA,
            pltpu.SemaphoreType.DMA,
        ],
    )
    def sc_kernel(x_hbm, hp_hbm, table_hbm, out_hbm, x_v, rows_v, tab_v, hp_v,
                  xsem, ssem):
        wid = lax.axis_index("s") * _NC + lax.axis_index("c")
        wbase = wid * per_w

        pltpu.sync_copy(hp_hbm, hp_v)
        pltpu.sync_copy(table_hbm, tab_v)
        pltpu.async_copy(
            x_hbm.at[pl.ds(pl.multiple_of(wbase, CHUNK), CHUNK)],
            x_v.at[pl.ds(0, CHUNK)],
            xsem,
        )

        def ebase(t):
            return pl.multiple_of(wbase + t * CHUNK, CHUNK)

        def wait_store(b):
            pltpu.make_async_copy(
                rows_v.at[b], out_hbm.at[pl.ds(0, CHUNK)], ssem
            ).wait()

        def group_body(g, carry):
            for b in range(2):
                t = g * 2 + b

                pl.when(g >= 1)(functools.partial(wait_store, b))

                pltpu.make_async_copy(
                    x_hbm.at[pl.ds(0, CHUNK)], x_v.at[pl.ds(b * CHUNK, CHUNK)], xsem
                ).wait()

                def prefetch():
                    pltpu.async_copy(
                        x_hbm.at[pl.ds(ebase(t + 1), CHUNK)],
                        x_v.at[pl.ds((1 - b) * CHUNK, CHUNK)],
                        xsem,
                    )
                if b == 0:
                    prefetch()
                else:
                    pl.when(g < n_groups - 1)(prefetch)

                for blk in range(CHUNK // 16):
                    xv = x_v[pl.ds(b * CHUNK + blk * 16, 16)]
                    t0 = (xv - MIN_VAL) * SCALE
                    gi = jnp.clip(t0.astype(jnp.int32), 0, N_BINS - 1)
                    hi = plsc.load_gather(hp_v, [gi + 1])
                    lo = plsc.load_gather(hp_v, [gi])
                    gi = gi + jnp.where(xv > hi, 1, 0) - jnp.where(xv <= lo, 1, 0)
                    gofs = gi * HIDDEN
                    for e in range(16):
                        base = gofs[e]
                        row = blk * 16 + e
                        for k in range(0, HIDDEN, 16):
                            rows_v[b, row, pl.ds(k, 16)] = (
                                tab_v[pl.ds(base + k, 16)]
                            )

                pltpu.async_copy(
                    rows_v.at[b], out_hbm.at[pl.ds(ebase(t), CHUNK)], ssem
                )
            return carry

        lax.fori_loop(0, n_groups, group_body, 0)

        wait_store(0)
        wait_store(1)

    return sc_kernel


def kernel(x, boundaries, table):
    n_rows, row_len = x.shape
    total = n_rows * row_len
    xf = x.reshape(total)
    hp = jnp.concatenate(
        [
            jnp.full((1,), -jnp.inf, jnp.float32),
            boundaries.astype(jnp.float32),
            jnp.full((HP_LEN - 1 - boundaries.shape[0],), jnp.inf, jnp.float32),
        ]
    )
    tab_flat = table.reshape(N_BINS * HIDDEN)
    out = _make_sc_call(total)(xf, hp, tab_flat)
    return out.reshape(n_rows, row_len, HIDDEN)
